# baseline (speedup 1.0000x reference)
import jax
import jax.numpy as jnp
from jax.experimental import pallas as pl
from jax.experimental.pallas import tpu as pltpu

_FLOOR = 1e-6
_L = 256


def _pcen_kernel(alpha_ref, delta_ref, root_ref, w_ref, x_hbm, o_ref, xbuf, insem):
    c = pl.program_id(0)
    C = pl.num_programs(0)
    slot = jax.lax.rem(c, 2)
    nslot = jax.lax.rem(c + 1, 2)

    @pl.when(c == 0)
    def _():
        pltpu.make_async_copy(x_hbm.at[:, 0, :], xbuf.at[0], insem.at[0]).start()

    @pl.when(c + 1 < C)
    def _():
        pltpu.make_async_copy(
            x_hbm.at[:, c + 1, :], xbuf.at[nslot], insem.at[nslot]
        ).start()

    pltpu.make_async_copy(x_hbm.at[:, c, :], xbuf.at[slot], insem.at[slot]).wait()

    w = jnp.clip(w_ref[c], 0.0, 1.0)
    a = jnp.minimum(alpha_ref[c], 1.0)
    d = delta_ref[c]
    inv_r = 1.0 / jnp.maximum(root_ref[c], 1.0)
    lw = jnp.maximum(jnp.log2(jnp.maximum(1.0 - w, 1e-45)), -1e4)

    L = _L
    B, T = o_ref.shape[1], o_ref.shape[2]
    nfull = T // L
    P = T - nfull * L

    k_idx = jax.lax.broadcasted_iota(jnp.int32, (L, L), 0)
    t_idx = jax.lax.broadcasted_iota(jnp.int32, (L, L), 1)
    e = (t_idx - k_idx).astype(jnp.float32)
    AT = jnp.where(e >= 0.0, w * jnp.exp2(e * lw), 0.0).astype(jnp.bfloat16)
    t_row = jax.lax.broadcasted_iota(jnp.int32, (1, L), 1).astype(jnp.float32)
    D = jnp.exp2((t_row + 1.0) * lw)
    d_pow = jnp.exp2(inv_r * jnp.log2(jnp.maximum(d, 1e-45)))

    carry = xbuf[slot, :, 0:1]

    def chunk(lo, ln, carry):
        Xj = xbuf[slot, :, lo : lo + ln]
        M = jax.lax.dot(
            Xj.astype(jnp.bfloat16), AT[:ln, :ln], preferred_element_type=jnp.float32
        )
        E = M + carry * D[:, :ln]
        denom = jnp.exp2(-a * jnp.log2(_FLOOR + E))
        base = Xj * denom + d
        o_ref[0, :, lo : lo + ln] = jnp.exp2(inv_r * jnp.log2(base)) - d_pow
        return E[:, ln - 1 : ln]

    for j in range(nfull):
        carry = chunk(j * L, L, carry)
    if P:
        chunk(nfull * L, P, carry)


def kernel(x, alpha, delta, root, ema_w):
    B, C, T = x.shape
    grid_spec = pltpu.PrefetchScalarGridSpec(
        num_scalar_prefetch=4,
        grid=(C,),
        in_specs=[pl.BlockSpec(memory_space=pl.ANY)],
        out_specs=pl.BlockSpec((1, B, T), lambda c, *_: (c, 0, 0)),
        scratch_shapes=[
            pltpu.VMEM((2, B, T), jnp.float32),
            pltpu.SemaphoreType.DMA((2,)),
        ],
    )
    out_t = pl.pallas_call(
        _pcen_kernel,
        grid_spec=grid_spec,
        out_shape=jax.ShapeDtypeStruct((C, B, T), x.dtype),
        compiler_params=pltpu.CompilerParams(
            dimension_semantics=("arbitrary",),
        ),
        name="pcen",
    )(alpha, delta, root, ema_w, x)
    return jnp.transpose(out_t, (1, 0, 2))


# speedup vs baseline: 2.0509x; 2.0509x over previous
import jax
import jax.numpy as jnp
from jax.experimental import pallas as pl
from jax.experimental.pallas import tpu as pltpu

_FLOOR = 1e-6
_L = 200
_NCORES = 2


def _pcen_kernel(alpha_ref, delta_ref, root_ref, w_ref, x_ref, o_ref):
    c = pl.program_id(0)
    w = jnp.clip(w_ref[c], 0.0, 1.0)
    a = jnp.minimum(alpha_ref[c], 1.0)
    d = delta_ref[c]
    inv_r = 1.0 / jnp.maximum(root_ref[c], 1.0)
    lw = jnp.maximum(jnp.log2(jnp.maximum(1.0 - w, 1e-45)), -1e4)

    L = _L
    T = x_ref.shape[1]

    t_idx = jax.lax.broadcasted_iota(jnp.int32, (L, L), 0)
    k_idx = jax.lax.broadcasted_iota(jnp.int32, (L, L), 1)
    e = (t_idx - k_idx).astype(jnp.float32)
    A = jnp.where(e >= 0.0, w * jnp.exp2(e * lw), 0.0)
    A16 = A.astype(jnp.bfloat16)
    t_col = jax.lax.broadcasted_iota(jnp.int32, (L, 1), 0).astype(jnp.float32)
    D = jnp.exp2((t_col + 1.0) * lw)
    d_pow = jnp.exp2(inv_r * jnp.log2(jnp.maximum(d, 1e-45)))

    carry = x_ref[0, 0:1, :].astype(jnp.float32)

    for j in range(T // L):
        X16 = x_ref[0, j * L : (j + 1) * L, :]
        M = jax.lax.dot(A16, X16, preferred_element_type=jnp.float32)
        E = M + D * carry
        denom = jnp.exp2(-a * jnp.log2(_FLOOR + E))
        base = X16.astype(jnp.float32) * denom + d
        o_ref[0, j * L : (j + 1) * L, :] = jnp.exp2(inv_r * jnp.log2(base)) - d_pow
        carry = E[L - 1 : L, :]


def kernel(x, alpha, delta, root, ema_w):
    B, C, T = x.shape
    xt = jnp.transpose(x, (1, 2, 0)).astype(jnp.bfloat16)
    grid_spec = pltpu.PrefetchScalarGridSpec(
        num_scalar_prefetch=4,
        grid=(C,),
        in_specs=[pl.BlockSpec((1, T, B), lambda c, *_: (c, 0, 0))],
        out_specs=pl.BlockSpec((1, T, B), lambda c, *_: (c, 0, 0)),
    )
    out_t = pl.pallas_call(
        _pcen_kernel,
        grid_spec=grid_spec,
        out_shape=jax.ShapeDtypeStruct((C, T, B), x.dtype),
        compiler_params=pltpu.CompilerParams(
            dimension_semantics=("parallel",),
        ),
        name="pcen",
    )(alpha, delta, root, ema_w, xt)
    return jnp.transpose(out_t, (2, 0, 1))


# speedup vs baseline: 3.1128x; 1.5178x over previous
import jax
import jax.numpy as jnp
from jax.experimental import pallas as pl
from jax.experimental.pallas import tpu as pltpu

_FLOOR = 1e-6
_L = 200
_NCORES = 2


def _pcen_kernel(alpha_ref, delta_ref, root_ref, w_ref, x_ref, o_ref):
    c = pl.program_id(0)
    w = jnp.clip(w_ref[c], 0.0, 1.0)
    a = jnp.minimum(alpha_ref[c], 1.0)
    d = delta_ref[c]
    inv_r = 1.0 / jnp.maximum(root_ref[c], 1.0)
    lw = jnp.maximum(jnp.log2(jnp.maximum(1.0 - w, 1e-45)), -1e4)

    L = _L
    T = x_ref.shape[1]

    t_idx = jax.lax.broadcasted_iota(jnp.int32, (L, L), 0)
    k_idx = jax.lax.broadcasted_iota(jnp.int32, (L, L), 1)
    e = (t_idx - k_idx).astype(jnp.float32)
    A = jnp.where(e >= 0.0, w * jnp.exp2(e * lw), 0.0)
    A16 = A.astype(jnp.bfloat16)
    t_col = jax.lax.broadcasted_iota(jnp.int32, (L, 1), 0).astype(jnp.float32)
    D = jnp.exp2((t_col + 1.0) * lw)
    d_pow = jnp.exp2(inv_r * jnp.log2(jnp.maximum(d, 1e-45)))

    carry = x_ref[0, 0:1, :]

    for j in range(T // L):
        Xj = x_ref[0, j * L : (j + 1) * L, :]
        M = jax.lax.dot(
            A16, Xj.astype(jnp.bfloat16), preferred_element_type=jnp.float32
        )
        E = M + D * carry
        denom = jnp.exp2(-a * jnp.log2(_FLOOR + E))
        base = Xj * denom + d
        o_ref[0, j * L : (j + 1) * L, :] = jnp.exp2(inv_r * jnp.log2(base)) - d_pow
        carry = E[L - 1 : L, :]


def kernel(x, alpha, delta, root, ema_w):
    B, C, T = x.shape
    xt = jnp.transpose(x, (1, 2, 0))
    grid_spec = pltpu.PrefetchScalarGridSpec(
        num_scalar_prefetch=4,
        grid=(C,),
        in_specs=[pl.BlockSpec((1, T, B), lambda c, *_: (c, 0, 0))],
        out_specs=pl.BlockSpec((1, T, B), lambda c, *_: (c, 0, 0)),
    )
    out_t = pl.pallas_call(
        _pcen_kernel,
        grid_spec=grid_spec,
        out_shape=jax.ShapeDtypeStruct((C, T, B), x.dtype),
        compiler_params=pltpu.CompilerParams(
            dimension_semantics=("parallel",),
        ),
        name="pcen",
    )(alpha, delta, root, ema_w, xt)
    return jnp.transpose(out_t, (2, 0, 1))
